# Initial kernel scaffold; baseline (speedup 1.0000x reference)
import jax
import jax.numpy as jnp
from jax.experimental import pallas as pl


def kernel(h, x, t, edge_index, msgh_W1, msgh_b1, msgh_W2, msgh_b2, attn_W, attn_b, upd_W1, upd_b1, upd_W2, upd_b2, msgx_W1, msgx_b1, msgx_W2, msgx_b2, msgx_W3, msgx_b3):
    raise NotImplementedError("write your pallas kernel here")


# speedup vs baseline: 1.5488x; 1.5488x over previous
import functools

import jax
import jax.numpy as jnp
from jax import lax
from jax.experimental import pallas as pl
from jax.experimental.pallas import tpu as pltpu
from jax.experimental.pallas import tpu_sc as plsc

_N = 10000
_E = 320000
_D = 128
_L = 4

_NC = 2
_NS = 16
_NW = _NC * _NS
_C = 80
_EW = _E // _NW
_NCHUNK = _EW // _C
_ET = _E // _NS
_ST = 624
_ZB = 48

_BE = 1280
_BN = 2000

_f32 = jnp.float32


def _silu(v):
    return v * jax.nn.sigmoid(v)


def _tc_pre_body(hh_ref, wd_ref, ws_ref, td_ref, ts_ref):
    hh = hh_ref[...]
    td_ref[...] = jnp.dot(hh, wd_ref[...], preferred_element_type=_f32)
    ts_ref[...] = jnp.dot(hh, ws_ref[...], preferred_element_type=_f32)


def _tc_pre(hh, wd, ws):
    grid = (_N // _BN,)
    return pl.pallas_call(
        _tc_pre_body,
        grid=grid,
        in_specs=[
            pl.BlockSpec((_BN, _D), lambda i: (i, 0)),
            pl.BlockSpec((_D, 2 * _D), lambda i: (0, 0)),
            pl.BlockSpec((_D, 2 * _D), lambda i: (0, 0)),
        ],
        out_specs=[
            pl.BlockSpec((_BN, 2 * _D), lambda i: (i, 0)),
            pl.BlockSpec((_BN, 2 * _D), lambda i: (i, 0)),
        ],
        out_shape=[
            jax.ShapeDtypeStruct((_N, 2 * _D), _f32),
            jax.ShapeDtypeStruct((_N, 2 * _D), _f32),
        ],
    )(hh, wd, ws)


def _sc_gather_body(td_hbm, ts_hbm, x4_hbm, dst_hbm, src_hbm,
                    z_out, geo_out,
                    dstv, srcv, zb, zb2, xtab, xb, sem1, sem2):
    wid = lax.axis_index("s") * _NC + lax.axis_index("c")
    base = wid * _EW

    pltpu.sync_copy(x4_hbm, xtab)
    zero16 = jnp.zeros((16,), _f32)

    def zrow(r, carry):
        xb[r, pl.ds(0, 16)] = zero16
        return carry

    lax.fori_loop(0, _C, zrow, 0)
    lanes = lax.iota(jnp.int32, 16)

    def chunk(i, carry):
        off = base + i * _C
        pltpu.sync_copy(dst_hbm.at[pl.ds(off, _C)], dstv)
        pltpu.sync_copy(src_hbm.at[pl.ds(off, _C)], srcv)
        c1 = pltpu.async_copy(td_hbm.at[dstv], zb, sem1)
        c2 = pltpu.async_copy(ts_hbm.at[srcv], zb2, sem2)
        for j in range(_C // 16):
            dv = dstv[pl.ds(j * 16, 16)]
            sv = srcv[pl.ds(j * 16, 16)]
            rows = lanes + (j * 16)
            d2v = jnp.zeros((16,), _f32)
            drow = lax.shift_right_logical(dv, 5)
            dcol = lax.shift_left(jnp.bitwise_and(dv, 31), 2)
            srow = lax.shift_right_logical(sv, 5)
            scol = lax.shift_left(jnp.bitwise_and(sv, 31), 2)
            for col in range(3):
                cvec = jnp.full((16,), col, jnp.int32)
                xi = plsc.load_gather(xtab, [drow, dcol + col])
                xj = plsc.load_gather(xtab, [srow, scol + col])
                df = xi - xj
                d2v = d2v + df * df
                plsc.store_scatter(xb, [rows, cvec], df)
            plsc.store_scatter(xb, [rows, jnp.full((16,), 3, jnp.int32)], d2v)
        c1.wait()
        c2.wait()

        def arow(r, carry):
            def acol(cc, c3):
                sl = pl.ds(cc * 16, 16)
                zb[r, sl] = zb[r, sl] + zb2[r, sl]
                return c3
            lax.fori_loop(0, 16, acol, 0)
            return carry

        lax.fori_loop(0, _C, arow, 0)
        plsc.subcore_barrier()
        pltpu.sync_copy(zb, z_out.at[pl.ds(off, _C)])
        pltpu.sync_copy(xb, geo_out.at[pl.ds(off, _C)])
        return carry

    lax.fori_loop(0, _NCHUNK, chunk, 0)


def _sc_gather(td, ts, x4, dst, src):
    mesh = plsc.VectorSubcoreMesh(core_axis_name="c", subcore_axis_name="s",
                                  num_cores=_NC, num_subcores=_NS)
    fn = pl.kernel(
        _sc_gather_body,
        out_type=[
            jax.ShapeDtypeStruct((_E, 2 * _D), _f32),
            jax.ShapeDtypeStruct((_E, 16), _f32),
        ],
        mesh=mesh,
        scratch_types=[
            pltpu.VMEM((_C,), jnp.int32),
            pltpu.VMEM((_C,), jnp.int32),
            pltpu.VMEM((_C, 2 * _D), _f32),
            pltpu.VMEM((_C, 2 * _D), _f32),
            pltpu.VMEM((320, 128), _f32),
            pltpu.VMEM((_C, 16), _f32),
            pltpu.SemaphoreType.DMA,
            pltpu.SemaphoreType.DMA,
        ],
        compiler_params=pltpu.CompilerParams(needs_layout_passes=False),
    )
    return fn(td, ts, x4, dst, src)


def _tc_edge_body(z_ref, geo_ref,
                  w1h_ref, b1h_ref, w2h_ref, b2h_ref, attw_ref, attb_ref,
                  w1x_ref, b1x_ref, w2x_ref, b2x_ref, w3_ref, b3_ref,
                  msgh_ref, msgx_ref):
    geo = geo_ref[...]
    lane = lax.broadcasted_iota(jnp.int32, geo.shape, 1)
    diff = jnp.where(lane < 3, geo, 0.0)
    d2 = geo[:, 3:4]
    d = jnp.sqrt(d2 + 1e-12)
    z = z_ref[...]
    zh = z[:, :_D] + d2 * w1h_ref[...] + b1h_ref[...]
    m = _silu(zh)
    m2 = _silu(jnp.dot(m, w2h_ref[...], preferred_element_type=_f32) + b2h_ref[...])
    attn = jax.nn.sigmoid(
        jnp.sum(m2 * attw_ref[...], axis=1, keepdims=True) + attb_ref[:, :1])
    msgh_ref[...] = attn * m2
    zx = z[:, _D:] + d2 * w1x_ref[...] + b1x_ref[...]
    mx = _silu(zx)
    mx2 = _silu(jnp.dot(mx, w2x_ref[...], preferred_element_type=_f32) + b2x_ref[...])
    s = jnp.sum(mx2 * w3_ref[...], axis=1, keepdims=True) + b3_ref[:, :1]
    msgx_ref[...] = jnp.pad(diff * (s / (d + 1.0)), ((0, 0), (0, _D - 16)))


def _tc_edge(z, geo, w1h, b1h, w2h, b2h, attw, attb, w1x, b1x, w2x, b2x, w3, b3):
    grid = (_E // _BE,)
    full = lambda shape: pl.BlockSpec(shape, lambda i: (0, 0))
    return pl.pallas_call(
        _tc_edge_body,
        grid=grid,
        in_specs=[
            pl.BlockSpec((_BE, 2 * _D), lambda i: (i, 0)),
            pl.BlockSpec((_BE, 16), lambda i: (i, 0)),
            full((1, _D)), full((1, _D)), full((_D, _D)), full((1, _D)),
            full((1, _D)), full((1, _D)),
            full((1, _D)), full((1, _D)), full((_D, _D)), full((1, _D)),
            full((1, _D)), full((1, _D)),
        ],
        out_specs=[
            pl.BlockSpec((_BE, _D), lambda i: (i, 0)),
            pl.BlockSpec((_BE, _D), lambda i: (i, 0)),
        ],
        out_shape=[
            jax.ShapeDtypeStruct((_E, _D), _f32),
            jax.ShapeDtypeStruct((_E, _D), _f32),
        ],
    )(z, geo, w1h, b1h, w2h, b2h, attw, attb, w1x, b1x, w2x, b2x, w3, b3)


def _sc_scatter_body(msgh_hbm, msgx_hbm, dst_hbm,
                     ah_out, ax_out,
                     agg, dstv, mh, zbuf, sem1):
    sid = lax.axis_index("s")
    start = sid * _ST
    zero16 = jnp.zeros((16,), _f32)

    def zero_fill():
        def zrow(r, carry):
            def zcol(cc, c2):
                zbuf[r, pl.ds(cc * 16, 16)] = zero16
                return c2
            lax.fori_loop(0, 8, zcol, 0)
            return carry

        lax.fori_loop(0, _ZB, zrow, 0)
        plsc.subcore_barrier()
        for k in range(_ST // _ZB):
            rows = pl.ds(start + k * _ZB, _ZB)
            pltpu.sync_copy(zbuf, agg.at[rows])

        @pl.when(sid == _NS - 1)
        def _():
            rows = pl.ds(start + _ST, 16)
            pltpu.sync_copy(zbuf.at[pl.ds(0, 16)], agg.at[rows])

        plsc.subcore_barrier()

    def accumulate(msg_hbm):
        base = sid * _ET

        def chunk(i, carry):
            off = base + i * _C
            pltpu.sync_copy(dst_hbm.at[pl.ds(off, _C)], dstv)
            c1 = pltpu.async_copy(msg_hbm.at[pl.ds(off, _C)], mh, sem1)
            c1.wait()
            pltpu.sync_copy(mh, agg.at[dstv], add=True)
            return carry

        lax.fori_loop(0, _ET // _C, chunk, 0)
        plsc.subcore_barrier()

    def readout(out_hbm):
        for k in range(_ST // _ZB):
            rows = pl.ds(start + k * _ZB, _ZB)
            pltpu.sync_copy(agg.at[rows], zbuf)
            pltpu.sync_copy(zbuf, out_hbm.at[rows])

        @pl.when(sid == _NS - 1)
        def _():
            rows = pl.ds(start + _ST, 16)
            pltpu.sync_copy(agg.at[rows], zbuf.at[pl.ds(0, 16)])
            pltpu.sync_copy(zbuf.at[pl.ds(0, 16)], out_hbm.at[rows])

        plsc.subcore_barrier()

    zero_fill()
    accumulate(msgh_hbm)
    readout(ah_out)
    zero_fill()
    accumulate(msgx_hbm)
    readout(ax_out)


def _sc_scatter(msgh, msgx, dst):
    mesh = plsc.VectorSubcoreMesh(core_axis_name="c", subcore_axis_name="s",
                                  num_cores=1, num_subcores=_NS)
    fn = pl.kernel(
        _sc_scatter_body,
        out_type=[
            jax.ShapeDtypeStruct((_N, _D), _f32),
            jax.ShapeDtypeStruct((_N, _D), _f32),
        ],
        mesh=mesh,
        scratch_types=[
            pltpu.VMEM_SHARED((_N, _D), _f32),
            pltpu.VMEM((_C,), jnp.int32),
            pltpu.VMEM((_C, _D), _f32),
            pltpu.VMEM((_ZB, _D), _f32),
            pltpu.SemaphoreType.DMA,
        ],
        compiler_params=pltpu.CompilerParams(needs_layout_passes=False),
    )
    return fn(msgh, msgx, dst)


def _tc_node_body(hh_ref, xt_ref, ah_ref, ax_ref,
                  u1a_ref, u1b_ref, b1_ref, u2_ref, b2_ref, wd_ref, ws_ref,
                  hh2_ref, xt2_ref, td_ref, ts_ref):
    hh = hh_ref[...]
    ah = ah_ref[...]
    tmp = _silu(jnp.dot(hh, u1a_ref[...], preferred_element_type=_f32)
                + jnp.dot(ah, u1b_ref[...], preferred_element_type=_f32)
                + b1_ref[...])
    hh2 = hh + jnp.dot(tmp, u2_ref[...], preferred_element_type=_f32) + b2_ref[...]
    hh2_ref[...] = hh2
    xt2_ref[...] = xt_ref[...] + ax_ref[...]
    td_ref[...] = jnp.dot(hh2, wd_ref[...], preferred_element_type=_f32)
    ts_ref[...] = jnp.dot(hh2, ws_ref[...], preferred_element_type=_f32)


def _tc_node(hh, xt, ah2, ax2, u1a, u1b, b1, u2, b2, wd, ws):
    grid = (_N // _BN,)
    full = lambda shape: pl.BlockSpec(shape, lambda i: (0, 0))
    return pl.pallas_call(
        _tc_node_body,
        grid=grid,
        in_specs=[
            pl.BlockSpec((_BN, _D), lambda i: (i, 0)),
            pl.BlockSpec((_BN, 16), lambda i: (i, 0)),
            pl.BlockSpec((_BN, _D), lambda i: (i, 0)),
            pl.BlockSpec((_BN, 16), lambda i: (i, 0)),
            full((_D, _D)), full((_D, _D)), full((1, _D)), full((_D, _D)),
            full((1, _D)), full((_D, 2 * _D)), full((_D, 2 * _D)),
        ],
        out_specs=[
            pl.BlockSpec((_BN, _D), lambda i: (i, 0)),
            pl.BlockSpec((_BN, 16), lambda i: (i, 0)),
            pl.BlockSpec((_BN, 2 * _D), lambda i: (i, 0)),
            pl.BlockSpec((_BN, 2 * _D), lambda i: (i, 0)),
        ],
        out_shape=[
            jax.ShapeDtypeStruct((_N, _D), _f32),
            jax.ShapeDtypeStruct((_N, 16), _f32),
            jax.ShapeDtypeStruct((_N, 2 * _D), _f32),
            jax.ShapeDtypeStruct((_N, 2 * _D), _f32),
        ],
    )(hh, xt, ah2, ax2, u1a, u1b, b1, u2, b2, wd, ws)


def _tc_node3_body(hh_ref, xt_ref, ah_ref, ax_ref,
                   u1a_ref, u1b_ref, b1_ref, u2_ref, b2_ref,
                   hh2_ref, xt2_ref, xsum_ref):
    i = pl.program_id(0)
    hh = hh_ref[...]
    ah = ah_ref[...]
    tmp = _silu(jnp.dot(hh, u1a_ref[...], preferred_element_type=_f32)
                + jnp.dot(ah, u1b_ref[...], preferred_element_type=_f32)
                + b1_ref[...])
    hh2 = hh + jnp.dot(tmp, u2_ref[...], preferred_element_type=_f32) + b2_ref[...]
    hh2_ref[...] = hh2
    xt2 = xt_ref[...] + ax_ref[...]
    xt2_ref[...] = xt2

    @pl.when(i == 0)
    def _():
        xsum_ref[...] = jnp.zeros_like(xsum_ref)

    xsum_ref[...] += jnp.sum(xt2, axis=0, keepdims=True)


def _tc_node3(hh, xt, ah2, ax2, u1a, u1b, b1, u2, b2):
    grid = (_N // _BN,)
    full = lambda shape: pl.BlockSpec(shape, lambda i: (0, 0))
    return pl.pallas_call(
        _tc_node3_body,
        grid=grid,
        in_specs=[
            pl.BlockSpec((_BN, _D), lambda i: (i, 0)),
            pl.BlockSpec((_BN, 16), lambda i: (i, 0)),
            pl.BlockSpec((_BN, _D), lambda i: (i, 0)),
            pl.BlockSpec((_BN, 16), lambda i: (i, 0)),
            full((_D, _D)), full((_D, _D)), full((1, _D)), full((_D, _D)),
            full((1, _D)),
        ],
        out_specs=[
            pl.BlockSpec((_BN, _D), lambda i: (i, 0)),
            pl.BlockSpec((_BN, 16), lambda i: (i, 0)),
            pl.BlockSpec((1, 16), lambda i: (0, 0)),
        ],
        out_shape=[
            jax.ShapeDtypeStruct((_N, _D), _f32),
            jax.ShapeDtypeStruct((_N, 16), _f32),
            jax.ShapeDtypeStruct((1, 16), _f32),
        ],
    )(hh, xt, ah2, ax2, u1a, u1b, b1, u2, b2)


def _tc_center_body(xt_ref, xsum_ref, xc_ref):
    xc_ref[...] = xt_ref[...] - xsum_ref[...] * (1.0 / _N)


def _tc_center(xt, xsum):
    grid = (_N // _BN,)
    return pl.pallas_call(
        _tc_center_body,
        grid=grid,
        in_specs=[
            pl.BlockSpec((_BN, 16), lambda i: (i, 0)),
            pl.BlockSpec((1, 16), lambda i: (0, 0)),
        ],
        out_specs=pl.BlockSpec((_BN, 16), lambda i: (i, 0)),
        out_shape=jax.ShapeDtypeStruct((_N, 16), _f32),
    )(xt, xsum)


def kernel(h, x, t, edge_index,
           msgh_W1, msgh_b1, msgh_W2, msgh_b2,
           attn_W, attn_b,
           upd_W1, upd_b1, upd_W2, upd_b2,
           msgx_W1, msgx_b1, msgx_W2, msgx_b2, msgx_W3, msgx_b3):
    src = edge_index[0]
    dst = edge_index[1]
    hh = jnp.concatenate([h, jnp.broadcast_to(t, (_N, 1)).astype(_f32)], axis=1)
    xt = jnp.zeros((_N, 16), _f32).at[:, :3].set(x)

    for l in range(_L):
        wd = jnp.concatenate([msgh_W1[l][:_D], msgx_W1[l][:_D]], axis=1)
        ws = jnp.concatenate([msgh_W1[l][_D:2 * _D], msgx_W1[l][_D:2 * _D]], axis=1)
        if l == 0:
            td, ts = _tc_pre(hh, wd, ws)
        x4 = jnp.zeros((10240, 4), _f32).at[:_N].set(xt[:, :4]).reshape(320, 128)
        z, geo = _sc_gather(td, ts, x4, dst, src)
        msgh, msgx = _tc_edge(
            z, geo,
            msgh_W1[l][2 * _D][None, :], msgh_b1[l][None, :],
            msgh_W2[l], msgh_b2[l][None, :],
            attn_W[l][:, 0][None, :], jnp.broadcast_to(attn_b[l][0], (1, _D)),
            msgx_W1[l][2 * _D][None, :], msgx_b1[l][None, :],
            msgx_W2[l], msgx_b2[l][None, :],
            msgx_W3[l][:, 0][None, :], jnp.broadcast_to(msgx_b3[l][0], (1, _D)),
        )
        ah2, ax128 = _sc_scatter(msgh, msgx, dst)
        ax2 = ax128[:, :16]
        if l < _L - 1:
            wd_n = jnp.concatenate([msgh_W1[l + 1][:_D], msgx_W1[l + 1][:_D]], axis=1)
            ws_n = jnp.concatenate([msgh_W1[l + 1][_D:2 * _D],
                                    msgx_W1[l + 1][_D:2 * _D]], axis=1)
            hh, xt, td, ts = _tc_node(
                hh, xt, ah2, ax2,
                upd_W1[l][:_D], upd_W1[l][_D:], upd_b1[l][None, :],
                upd_W2[l], upd_b2[l][None, :], wd_n, ws_n)
        else:
            hh, xt, xsum = _tc_node3(
                hh, xt, ah2, ax2,
                upd_W1[l][:_D], upd_W1[l][_D:], upd_b1[l][None, :],
                upd_W2[l], upd_b2[l][None, :])

    xc = _tc_center(xt, xsum)
    return jnp.concatenate([xc[:, :3], hh[:, :127]], axis=1)
